# baseline (speedup 1.0000x reference)
import functools

import jax
import jax.numpy as jnp
from jax import lax
from jax.experimental import pallas as pl
from jax.experimental.pallas import tpu as pltpu
from jax.experimental.pallas import tpu_sc as plsc

N = 1048576
NBLK = N // 128
LANES = 16
B = 256
BBLK = B // 128
NG = B // LANES
IDX_CHUNK = 128

PLANE_ROWS = 2048 * 2048 // 8

_COLMAP = []
for _c in range(3):
    _COLMAP.append((0, _c, _c * 9 + 0))
    for _k in range(3):
        _COLMAP.append((1, 3 * _c + _k, _c * 9 + 1 + _k))
    for _k in range(5):
        _COLMAP.append((2, 5 * _c + _k, _c * 9 + 4 + _k))


def _scratch_set():
    return [
        pltpu.VMEM((BBLK, 2, 128), jnp.float32),
        pltpu.VMEM((3 * B,), jnp.int32),
        pltpu.VMEM((2 * B,), jnp.int32),
        pltpu.VMEM((2 * B,), jnp.int32),
        pltpu.VMEM((B,), jnp.int32),
        pltpu.VMEM((B,), jnp.int32),
        pltpu.VMEM((B,), jnp.int32),
        pltpu.VMEM((3 * B, 8), jnp.float32),
        pltpu.VMEM((2 * B, 16), jnp.float32),
        pltpu.VMEM((2 * B, 16), jnp.float32),
        pltpu.VMEM((9, BBLK, 4, 128), jnp.float32),
        pltpu.SemaphoreType.DMA,
        pltpu.SemaphoreType.DMA,
    ]


def kernel(uv_coords, tex0, tex1, tex2):
    t0_rows = (tex0.transpose(2, 0, 1)
               .reshape(3, 256, 8, 16, 128)
               .transpose(0, 1, 3, 2, 4)
               .reshape(3 * PLANE_ROWS, 8))
    tabs = [tex1.reshape(-1, 16), tex2.reshape(-1, 16)]
    maxrow = [t.shape[0] - 1 for t in tabs]
    uv_blk = uv_coords.reshape(NBLK, 128, 2).transpose(0, 2, 1)

    info = plsc.get_sparse_core_info()
    nc, ns = info.num_cores, info.num_subcores
    nw = nc * ns
    pts_per_tile = N // nw
    n_chunks = pts_per_tile // B
    n_pairs = n_chunks // 2

    @functools.partial(
        pl.kernel,
        out_type=jax.ShapeDtypeStruct((9, NBLK, 4, 128), jnp.float32),
        mesh=plsc.VectorSubcoreMesh(core_axis_name="c", subcore_axis_name="s"),
        compiler_params=pltpu.CompilerParams(
            needs_layout_passes=False, use_tc_tiling_on_sc=False),
        scratch_types=_scratch_set() + _scratch_set(),
    )
    def sc_kernel(uv_hbm, t0_hbm, t1_hbm, t2_hbm, out_hbm, *scr):
        bufA, bufB = scr[:13], scr[13:]
        wid = lax.axis_index("s") * nc + lax.axis_index("c")
        iota = lax.iota(jnp.int32, LANES)
        zeros = jnp.zeros((LANES,), jnp.int32)

        def chunk_base(ci):
            return wid * pts_per_tile + ci * B

        def fire(ci, buf):
            (uv_v, i0_v, i1_v, i2_v, bo0_v, bo1_v, bo2_v,
             g0_v, g1_v, g2_v, out_v, sem_g, sem_o) = buf
            blk0 = chunk_base(ci) // 128
            pltpu.sync_copy(uv_hbm.at[pl.ds(blk0, BBLK)], uv_v)

            @plsc.parallel_loop(0, NG, 1, unroll=2)
            def gen_body(g):
                q = iota + g * LANES
                qb = q >> 7
                ql = q & 127
                q2 = q << 1
                q3 = q2 + q
                u = plsc.load_gather(uv_v, [qb, zeros, ql])
                v = plsc.load_gather(uv_v, [qb, zeros + 1, ql])
                ix = jnp.clip((u * 2048.0).astype(jnp.int32), 0, 2047)
                iy = jnp.clip((v * 2048.0).astype(jnp.int32), 0, 2047)
                w = ((((iy >> 3) << 4) + (ix >> 7)) << 10) \
                    + ((iy & 7) << 7) + (ix & 127)
                r0 = w >> 3
                plsc.store_scatter(i0_v, [q3], r0)
                plsc.store_scatter(i0_v, [q3 + 1], r0 + PLANE_ROWS)
                plsc.store_scatter(i0_v, [q3 + 2], r0 + 2 * PLANE_ROWS)
                plsc.store_scatter(bo0_v, [q], ix & 7)
                for s, d in ((0, 9), (1, 15)):
                    sh = s + 1
                    texel = ((iy >> sh) << (11 - sh)) + (ix >> sh)
                    o = texel * d
                    r = o >> 4
                    r2 = jnp.minimum(r + 1, maxrow[s])
                    iref = (i1_v, i2_v)[s]
                    plsc.store_scatter(iref, [q2], r)
                    plsc.store_scatter(iref, [q2 + 1], r2)
                    plsc.store_scatter((bo1_v, bo2_v)[s], [q],
                                       (q << 5) + (o & 15))

            for i in range(3 * B // IDX_CHUNK):
                sl = pl.ds(i * IDX_CHUNK, IDX_CHUNK)
                pltpu.async_copy(t0_hbm.at[i0_v.at[sl]], g0_v.at[sl], sem_g)
            for s in range(2):
                tab = (t1_hbm, t2_hbm)[s]
                iref = (i1_v, i2_v)[s]
                gref = (g1_v, g2_v)[s]
                for i in range(2 * B // IDX_CHUNK):
                    sl = pl.ds(i * IDX_CHUNK, IDX_CHUNK)
                    pltpu.async_copy(tab.at[iref.at[sl]], gref.at[sl], sem_g)

        def drain_gathers(buf):
            (uv_v, i0_v, i1_v, i2_v, bo0_v, bo1_v, bo2_v,
             g0_v, g1_v, g2_v, out_v, sem_g, sem_o) = buf
            for i in range(3 * B // IDX_CHUNK):
                sl = pl.ds(i * IDX_CHUNK, IDX_CHUNK)
                pltpu.make_async_copy(
                    t0_hbm.at[i0_v.at[sl]], g0_v.at[sl], sem_g).wait()
            for s in range(2):
                tab = (t1_hbm, t2_hbm)[s]
                iref = (i1_v, i2_v)[s]
                gref = (g1_v, g2_v)[s]
                for i in range(2 * B // IDX_CHUNK):
                    sl = pl.ds(i * IDX_CHUNK, IDX_CHUNK)
                    pltpu.make_async_copy(
                        tab.at[iref.at[sl]], gref.at[sl], sem_g).wait()

        def drain_out(ci, buf):
            out_v, sem_o = buf[10], buf[12]
            blk0 = chunk_base(ci) // 128
            pltpu.make_async_copy(
                out_v, out_hbm.at[:, pl.ds(blk0, BBLK)], sem_o).wait()

        def shuffle_and_out(ci, buf):
            (uv_v, i0_v, i1_v, i2_v, bo0_v, bo1_v, bo2_v,
             g0_v, g1_v, g2_v, out_v, sem_g, sem_o) = buf
            blk0 = chunk_base(ci) // 128

            @plsc.parallel_loop(0, NG, 1, unroll=4)
            def shuf_body(g):
                q = iota + g * LANES
                qb = q >> 7
                ql = q & 127
                q3 = (q << 1) + q
                x7 = plsc.load_gather(bo0_v, [q])
                bo1 = plsc.load_gather(bo1_v, [q])
                bo2 = plsc.load_gather(bo2_v, [q])
                for s, col, j in _COLMAP:
                    if s == 0:
                        val = plsc.load_gather(g0_v, [q3 + col, x7])
                    else:
                        w = (bo1, bo2)[s - 1] + col
                        val = plsc.load_gather(
                            (g1_v, g2_v)[s - 1], [w >> 4, w & 15])
                    plsc.store_scatter(
                        out_v,
                        [zeros + (j % 9), qb, zeros + (j // 9), ql], val)

            pltpu.async_copy(out_v, out_hbm.at[:, pl.ds(blk0, BBLK)], sem_o)

        fire(0, bufA)

        def pair_body(cp, _):
            ca = cp * 2
            cb = ca + 1
            fire(cb, bufB)
            drain_gathers(bufA)

            @pl.when(cp > 0)
            def _older_a():
                drain_out(ca, bufA)

            shuffle_and_out(ca, bufA)

            @pl.when(cp < n_pairs - 1)
            def _next_a():
                fire(ca + 2, bufA)

            drain_gathers(bufB)

            @pl.when(cp > 0)
            def _older_b():
                drain_out(cb, bufB)

            shuffle_and_out(cb, bufB)
            return _

        lax.fori_loop(0, n_pairs, pair_body, None)
        drain_out(0, bufA)
        drain_out(0, bufB)

    out = sc_kernel(uv_blk, t0_rows, *tabs)
    return (out.transpose(1, 3, 2, 0)
            .reshape(N, 4, 9)[:, :3, :])


# speedup vs baseline: 1.7152x; 1.7152x over previous
import functools

import jax
import jax.numpy as jnp
from jax import lax
from jax.experimental import pallas as pl
from jax.experimental.pallas import tpu as pltpu
from jax.experimental.pallas import tpu_sc as plsc

N = 1048576
NBLK = N // 128
LANES = 16
B = 256
BBLK = B // 128
NG = B // LANES
IDX_CHUNK = 128

PLANE_ROWS = 2048 * 2048 // 8

_T1 = (9, 128, 8)
_T2 = (15, 64, 4)

_COLMAP = []
for _c in range(3):
    _COLMAP.append((0, _c, _c * 9 + 0))
    for _k in range(3):
        _COLMAP.append((1, 3 * _c + _k, _c * 9 + 1 + _k))
    for _k in range(5):
        _COLMAP.append((2, 5 * _c + _k, _c * 9 + 4 + _k))


def _scratch_set():
    return [
        pltpu.VMEM((BBLK, 2, 128), jnp.float32),
        pltpu.VMEM((3 * B,), jnp.int32),
        pltpu.VMEM((B,), jnp.int32),
        pltpu.VMEM((B,), jnp.int32),
        pltpu.VMEM((B,), jnp.int32),
        pltpu.VMEM((3 * B, 8), jnp.float32),
        pltpu.VMEM((B, 16), jnp.float32),
        pltpu.VMEM((B, 16), jnp.float32),
        pltpu.VMEM((9, BBLK, 4, 128), jnp.float32),
        pltpu.SemaphoreType.DMA,
        pltpu.SemaphoreType.DMA,
    ]


def kernel(uv_coords, tex0, tex1, tex2):
    t0_rows = (tex0.transpose(2, 0, 1)
               .reshape(3, 256, 8, 16, 128)
               .transpose(0, 1, 3, 2, 4)
               .reshape(3 * PLANE_ROWS, 8))
    t1_planes = (tex1.transpose(2, 0, 1)
                 .reshape(9, 128, 8, 8, 128)
                 .transpose(0, 1, 3, 2, 4)
                 .reshape(9, 128, 8, 1024))
    t2_planes = (tex2.transpose(2, 0, 1)
                 .reshape(15, 64, 8, 4, 128)
                 .transpose(0, 1, 3, 2, 4)
                 .reshape(15, 64, 4, 1024))
    uv_blk = uv_coords.reshape(NBLK, 128, 2).transpose(0, 2, 1)

    info = plsc.get_sparse_core_info()
    nc, ns = info.num_cores, info.num_subcores
    nw = nc * ns
    pts_per_tile = N // nw
    n_chunks = pts_per_tile // B
    n_pairs = n_chunks // 2

    @functools.partial(
        pl.kernel,
        out_type=[
            jax.ShapeDtypeStruct((9, NBLK, 4, 128), jnp.float32),
            jax.ShapeDtypeStruct((1024 * 1024, 16), jnp.float32),
            jax.ShapeDtypeStruct((512 * 512, 16), jnp.float32),
        ],
        mesh=plsc.VectorSubcoreMesh(core_axis_name="c", subcore_axis_name="s"),
        compiler_params=pltpu.CompilerParams(
            needs_layout_passes=False, use_tc_tiling_on_sc=False),
        scratch_types=_scratch_set() + _scratch_set() + [
            pltpu.VMEM((15, 1024), jnp.float32),
            pltpu.VMEM((1024, 16), jnp.float32),
            pltpu.SemaphoreType.DMA,
        ],
    )
    def sc_kernel(uv_hbm, t0_hbm, t1pl_hbm, t2pl_hbm,
                  out_hbm, t1p_hbm, t2p_hbm, *scr):
        bufA, bufB = scr[:11], scr[11:22]
        rin_v, rout_v, rsem = scr[22:]
        wid = lax.axis_index("s") * nc + lax.axis_index("c")
        sid = lax.axis_index("s")
        iota = lax.iota(jnp.int32, LANES)
        zeros = jnp.zeros((LANES,), jnp.int32)

        def repack(planes_hbm, packed_hbm, nchan, nty, ntx):
            nblocks = nty * ntx
            per_tile = nblocks // ns
            texw = ntx * 128

            def blk_body(bi, _):
                ty = bi // ntx
                tx = bi - ty * ntx
                pltpu.sync_copy(planes_hbm.at[:, ty, tx],
                                rin_v.at[pl.ds(0, nchan)])

                @plsc.parallel_loop(0, 64, 1, unroll=2)
                def fill(g):
                    vrow = iota + g * LANES
                    for c in range(nchan):
                        val = rin_v[c, pl.ds(g * LANES, LANES)]
                        plsc.store_scatter(rout_v, [vrow, zeros + c], val)

                row0 = (ty * 8) * texw + tx * 128
                outs = []
                for sy in range(8):
                    outs.append(pltpu.async_copy(
                        rout_v.at[pl.ds(sy * 128, 128)],
                        packed_hbm.at[pl.ds(row0 + sy * texw, 128)], rsem))
                for cpy in outs:
                    cpy.wait()
                return _

            lax.fori_loop(sid * per_tile, (sid + 1) * per_tile,
                          blk_body, None)

        repack(t1pl_hbm, t1p_hbm, *_T1)
        repack(t2pl_hbm, t2p_hbm, *_T2)
        plsc.subcore_barrier()

        def chunk_base(ci):
            return wid * pts_per_tile + ci * B

        def fire(ci, buf):
            (uv_v, i0_v, i1_v, i2_v, bo0_v,
             g0_v, g1_v, g2_v, out_v, sem_g, sem_o) = buf
            blk0 = chunk_base(ci) // 128
            pltpu.sync_copy(uv_hbm.at[pl.ds(blk0, BBLK)], uv_v)

            @plsc.parallel_loop(0, NG, 1, unroll=2)
            def gen_body(g):
                q = iota + g * LANES
                qb = q >> 7
                ql = q & 127
                q2 = q << 1
                q3 = q2 + q
                u = plsc.load_gather(uv_v, [qb, zeros, ql])
                v = plsc.load_gather(uv_v, [qb, zeros + 1, ql])
                ix = jnp.clip((u * 2048.0).astype(jnp.int32), 0, 2047)
                iy = jnp.clip((v * 2048.0).astype(jnp.int32), 0, 2047)
                w = ((((iy >> 3) << 4) + (ix >> 7)) << 10) \
                    + ((iy & 7) << 7) + (ix & 127)
                r0 = w >> 3
                plsc.store_scatter(i0_v, [q3], r0)
                plsc.store_scatter(i0_v, [q3 + 1], r0 + PLANE_ROWS)
                plsc.store_scatter(i0_v, [q3 + 2], r0 + 2 * PLANE_ROWS)
                plsc.store_scatter(bo0_v, [q], ix & 7)
                plsc.store_scatter(i1_v, [q],
                                   ((iy >> 1) << 10) + (ix >> 1))
                plsc.store_scatter(i2_v, [q],
                                   ((iy >> 2) << 9) + (ix >> 2))

            for i in range(3 * B // IDX_CHUNK):
                sl = pl.ds(i * IDX_CHUNK, IDX_CHUNK)
                pltpu.async_copy(t0_hbm.at[i0_v.at[sl]], g0_v.at[sl], sem_g)
            for i in range(B // IDX_CHUNK):
                sl = pl.ds(i * IDX_CHUNK, IDX_CHUNK)
                pltpu.async_copy(t1p_hbm.at[i1_v.at[sl]], g1_v.at[sl], sem_g)
                pltpu.async_copy(t2p_hbm.at[i2_v.at[sl]], g2_v.at[sl], sem_g)

        def drain_gathers(buf):
            (uv_v, i0_v, i1_v, i2_v, bo0_v,
             g0_v, g1_v, g2_v, out_v, sem_g, sem_o) = buf
            for i in range(3 * B // IDX_CHUNK):
                sl = pl.ds(i * IDX_CHUNK, IDX_CHUNK)
                pltpu.make_async_copy(
                    t0_hbm.at[i0_v.at[sl]], g0_v.at[sl], sem_g).wait()
            for i in range(B // IDX_CHUNK):
                sl = pl.ds(i * IDX_CHUNK, IDX_CHUNK)
                pltpu.make_async_copy(
                    t1p_hbm.at[i1_v.at[sl]], g1_v.at[sl], sem_g).wait()
                pltpu.make_async_copy(
                    t2p_hbm.at[i2_v.at[sl]], g2_v.at[sl], sem_g).wait()

        def drain_out(ci, buf):
            out_v, sem_o = buf[8], buf[10]
            blk0 = chunk_base(ci) // 128
            pltpu.make_async_copy(
                out_v, out_hbm.at[:, pl.ds(blk0, BBLK)], sem_o).wait()

        def shuffle_and_out(ci, buf):
            (uv_v, i0_v, i1_v, i2_v, bo0_v,
             g0_v, g1_v, g2_v, out_v, sem_g, sem_o) = buf
            blk0 = chunk_base(ci) // 128

            @plsc.parallel_loop(0, NG, 1, unroll=2)
            def shuf_body(g):
                q = iota + g * LANES
                qb = q >> 7
                ql = q & 127
                q3 = (q << 1) + q
                x7 = plsc.load_gather(bo0_v, [q])
                for s, col, j in _COLMAP:
                    if s == 0:
                        val = plsc.load_gather(g0_v, [q3 + col, x7])
                    else:
                        val = plsc.load_gather(
                            (g1_v, g2_v)[s - 1], [q, zeros + col])
                    plsc.store_scatter(
                        out_v,
                        [zeros + (j % 9), qb, zeros + (j // 9), ql], val)

            pltpu.async_copy(out_v, out_hbm.at[:, pl.ds(blk0, BBLK)], sem_o)

        fire(0, bufA)

        def pair_body(cp, _):
            ca = cp * 2
            cb = ca + 1
            fire(cb, bufB)
            drain_gathers(bufA)

            @pl.when(cp > 0)
            def _older_a():
                drain_out(ca, bufA)

            shuffle_and_out(ca, bufA)

            @pl.when(cp < n_pairs - 1)
            def _next_a():
                fire(ca + 2, bufA)

            drain_gathers(bufB)

            @pl.when(cp > 0)
            def _older_b():
                drain_out(cb, bufB)

            shuffle_and_out(cb, bufB)
            return _

        lax.fori_loop(0, n_pairs, pair_body, None)
        drain_out(0, bufA)
        drain_out(0, bufB)

    out, _, _ = sc_kernel(uv_blk, t0_rows, t1_planes, t2_planes)
    return (out.transpose(1, 3, 2, 0)
            .reshape(N, 4, 9)[:, :3, :])


# speedup vs baseline: 2.0786x; 1.2119x over previous
import functools

import jax
import jax.numpy as jnp
from jax import lax
from jax.experimental import pallas as pl
from jax.experimental.pallas import tpu as pltpu
from jax.experimental.pallas import tpu_sc as plsc

N = 1048576
NBLK = N // 128
LANES = 16
B = 256
BBLK = B // 128
NG = B // LANES
IDX_CHUNK = 128

PLANE_ROWS = 2048 * 2048 // 8

_T1 = (9, 128, 8)
_T2 = (15, 64, 4)

_COLMAP = []
for _c in range(3):
    _COLMAP.append((0, _c, _c * 9 + 0))
    for _k in range(3):
        _COLMAP.append((1, 3 * _c + _k, _c * 9 + 1 + _k))
    for _k in range(5):
        _COLMAP.append((2, 5 * _c + _k, _c * 9 + 4 + _k))


def _scratch_set():
    return [
        pltpu.VMEM((BBLK, 2, 128), jnp.float32),
        pltpu.VMEM((3 * B,), jnp.int32),
        pltpu.VMEM((B,), jnp.int32),
        pltpu.VMEM((B,), jnp.int32),
        pltpu.VMEM((B,), jnp.int32),
        pltpu.VMEM((3 * B, 8), jnp.float32),
        pltpu.VMEM((B, 16), jnp.float32),
        pltpu.VMEM((B, 16), jnp.float32),
        pltpu.VMEM((9, BBLK, 4, 128), jnp.float32),
        pltpu.SemaphoreType.DMA,
        pltpu.SemaphoreType.DMA,
    ]


def kernel(uv_coords, tex0, tex1, tex2):
    t0_rows = (tex0.transpose(2, 0, 1)
               .reshape(3, 256, 8, 16, 128)
               .transpose(0, 1, 3, 2, 4)
               .reshape(3 * PLANE_ROWS, 8))
    t1_planes = (tex1.transpose(2, 0, 1)
                 .reshape(9, 128, 8, 8, 128)
                 .transpose(0, 1, 3, 2, 4)
                 .reshape(9, 128, 8, 1024))
    t2_planes = (tex2.transpose(2, 0, 1)
                 .reshape(15, 64, 8, 4, 128)
                 .transpose(0, 1, 3, 2, 4)
                 .reshape(15, 64, 4, 1024))
    uv_blk = uv_coords.reshape(NBLK, 128, 2).transpose(0, 2, 1)

    info = plsc.get_sparse_core_info()
    nc, ns = info.num_cores, info.num_subcores
    nw = nc * ns
    pts_per_tile = N // nw
    n_chunks = pts_per_tile // B
    n_pairs = n_chunks // 2

    @functools.partial(
        pl.kernel,
        out_type=[
            jax.ShapeDtypeStruct((9, NBLK, 4, 128), jnp.float32),
            jax.ShapeDtypeStruct((1024 * 1024, 16), jnp.float32),
            jax.ShapeDtypeStruct((512 * 512, 16), jnp.float32),
        ],
        mesh=plsc.VectorSubcoreMesh(core_axis_name="c", subcore_axis_name="s"),
        compiler_params=pltpu.CompilerParams(
            needs_layout_passes=False, use_tc_tiling_on_sc=False),
        scratch_types=_scratch_set() + _scratch_set() + [
            pltpu.VMEM((15, 1024), jnp.float32),
            pltpu.VMEM((15, 1024), jnp.float32),
            pltpu.VMEM((1024, 16), jnp.float32),
            pltpu.SemaphoreType.DMA,
            pltpu.SemaphoreType.DMA,
            pltpu.SemaphoreType.DMA,
        ],
    )
    def sc_kernel(uv_hbm, t0_hbm, t1pl_hbm, t2pl_hbm,
                  out_hbm, t1p_hbm, t2p_hbm, *scr):
        bufA, bufB = scr[:11], scr[11:22]
        rinA, rinB, rout_v, rsemA, rsemB, rosem = scr[22:]
        wid = lax.axis_index("s") * nc + lax.axis_index("c")
        sid = lax.axis_index("s")
        iota = lax.iota(jnp.int32, LANES)
        zeros = jnp.zeros((LANES,), jnp.int32)

        def repack(planes_hbm, packed_hbm, nchan, nty, ntx):
            nblocks = nty * ntx
            per_tile = nblocks // ns
            texw = ntx * 128
            lo = sid * per_tile

            def fire_in(bi, rin, rsem):
                ty = bi // ntx
                tx = bi - ty * ntx
                pltpu.async_copy(planes_hbm.at[:, ty, tx],
                                 rin.at[pl.ds(0, nchan)], rsem)

            def wait_in(bi, rin, rsem):
                ty = bi // ntx
                tx = bi - ty * ntx
                pltpu.make_async_copy(planes_hbm.at[:, ty, tx],
                                      rin.at[pl.ds(0, nchan)], rsem).wait()

            def process(bi, rin):
                ty = bi // ntx
                tx = bi - ty * ntx

                @plsc.parallel_loop(0, 64, 1, unroll=2)
                def fill(g):
                    vrow = iota + g * LANES
                    for c in range(nchan):
                        val = rin[c, pl.ds(g * LANES, LANES)]
                        plsc.store_scatter(rout_v, [vrow, zeros + c], val)

                row0 = (ty * 8) * texw + tx * 128
                outs = []
                for sy in range(8):
                    outs.append(pltpu.async_copy(
                        rout_v.at[pl.ds(sy * 128, 128)],
                        packed_hbm.at[pl.ds(row0 + sy * texw, 128)], rosem))
                for cpy in outs:
                    cpy.wait()

            fire_in(lo, rinA, rsemA)

            def rpair(cp, _):
                ba = lo + cp * 2
                bb = ba + 1
                fire_in(bb, rinB, rsemB)
                wait_in(ba, rinA, rsemA)
                process(ba, rinA)

                @pl.when(cp < per_tile // 2 - 1)
                def _next():
                    fire_in(ba + 2, rinA, rsemA)

                wait_in(bb, rinB, rsemB)
                process(bb, rinB)
                return _

            lax.fori_loop(0, per_tile // 2, rpair, None)

        repack(t1pl_hbm, t1p_hbm, *_T1)
        repack(t2pl_hbm, t2p_hbm, *_T2)
        plsc.subcore_barrier()

        def chunk_base(ci):
            return wid * pts_per_tile + ci * B

        def fire(ci, buf):
            (uv_v, i0_v, i1_v, i2_v, bo0_v,
             g0_v, g1_v, g2_v, out_v, sem_g, sem_o) = buf
            blk0 = chunk_base(ci) // 128
            pltpu.sync_copy(uv_hbm.at[pl.ds(blk0, BBLK)], uv_v)

            @plsc.parallel_loop(0, NG, 1, unroll=2)
            def gen_body(g):
                q = iota + g * LANES
                qb = q >> 7
                ql = q & 127
                q2 = q << 1
                q3 = q2 + q
                u = plsc.load_gather(uv_v, [qb, zeros, ql])
                v = plsc.load_gather(uv_v, [qb, zeros + 1, ql])
                ix = jnp.clip((u * 2048.0).astype(jnp.int32), 0, 2047)
                iy = jnp.clip((v * 2048.0).astype(jnp.int32), 0, 2047)
                w = ((((iy >> 3) << 4) + (ix >> 7)) << 10) \
                    + ((iy & 7) << 7) + (ix & 127)
                r0 = w >> 3
                plsc.store_scatter(i0_v, [q3], r0)
                plsc.store_scatter(i0_v, [q3 + 1], r0 + PLANE_ROWS)
                plsc.store_scatter(i0_v, [q3 + 2], r0 + 2 * PLANE_ROWS)
                plsc.store_scatter(bo0_v, [q], ix & 7)
                plsc.store_scatter(i1_v, [q],
                                   ((iy >> 1) << 10) + (ix >> 1))
                plsc.store_scatter(i2_v, [q],
                                   ((iy >> 2) << 9) + (ix >> 2))

            for i in range(3 * B // IDX_CHUNK):
                sl = pl.ds(i * IDX_CHUNK, IDX_CHUNK)
                pltpu.async_copy(t0_hbm.at[i0_v.at[sl]], g0_v.at[sl], sem_g)
            for i in range(B // IDX_CHUNK):
                sl = pl.ds(i * IDX_CHUNK, IDX_CHUNK)
                pltpu.async_copy(t1p_hbm.at[i1_v.at[sl]], g1_v.at[sl], sem_g)
                pltpu.async_copy(t2p_hbm.at[i2_v.at[sl]], g2_v.at[sl], sem_g)

        def drain_gathers(buf):
            (uv_v, i0_v, i1_v, i2_v, bo0_v,
             g0_v, g1_v, g2_v, out_v, sem_g, sem_o) = buf
            for i in range(3 * B // IDX_CHUNK):
                sl = pl.ds(i * IDX_CHUNK, IDX_CHUNK)
                pltpu.make_async_copy(
                    t0_hbm.at[i0_v.at[sl]], g0_v.at[sl], sem_g).wait()
            for i in range(B // IDX_CHUNK):
                sl = pl.ds(i * IDX_CHUNK, IDX_CHUNK)
                pltpu.make_async_copy(
                    t1p_hbm.at[i1_v.at[sl]], g1_v.at[sl], sem_g).wait()
                pltpu.make_async_copy(
                    t2p_hbm.at[i2_v.at[sl]], g2_v.at[sl], sem_g).wait()

        def drain_out(ci, buf):
            out_v, sem_o = buf[8], buf[10]
            blk0 = chunk_base(ci) // 128
            pltpu.make_async_copy(
                out_v, out_hbm.at[:, pl.ds(blk0, BBLK)], sem_o).wait()

        def shuffle_and_out(ci, buf):
            (uv_v, i0_v, i1_v, i2_v, bo0_v,
             g0_v, g1_v, g2_v, out_v, sem_g, sem_o) = buf
            blk0 = chunk_base(ci) // 128

            @plsc.parallel_loop(0, NG, 1, unroll=2)
            def shuf_body(g):
                q = iota + g * LANES
                qb = q >> 7
                ql = q & 127
                q3 = (q << 1) + q
                x7 = plsc.load_gather(bo0_v, [q])
                for s, col, j in _COLMAP:
                    if s == 0:
                        val = plsc.load_gather(g0_v, [q3 + col, x7])
                    else:
                        val = plsc.load_gather(
                            (g1_v, g2_v)[s - 1], [q, zeros + col])
                    plsc.store_scatter(
                        out_v,
                        [zeros + (j % 9), qb, zeros + (j // 9), ql], val)

            pltpu.async_copy(out_v, out_hbm.at[:, pl.ds(blk0, BBLK)], sem_o)

        fire(0, bufA)

        def pair_body(cp, _):
            ca = cp * 2
            cb = ca + 1
            fire(cb, bufB)
            drain_gathers(bufA)

            @pl.when(cp > 0)
            def _older_a():
                drain_out(ca, bufA)

            shuffle_and_out(ca, bufA)

            @pl.when(cp < n_pairs - 1)
            def _next_a():
                fire(ca + 2, bufA)

            drain_gathers(bufB)

            @pl.when(cp > 0)
            def _older_b():
                drain_out(cb, bufB)

            shuffle_and_out(cb, bufB)
            return _

        lax.fori_loop(0, n_pairs, pair_body, None)
        drain_out(0, bufA)
        drain_out(0, bufB)

    out, _, _ = sc_kernel(uv_blk, t0_rows, t1_planes, t2_planes)
    return (out.transpose(1, 3, 2, 0)
            .reshape(N, 4, 9)[:, :3, :])


# speedup vs baseline: 2.2053x; 1.0609x over previous
import functools

import jax
import jax.numpy as jnp
from jax import lax
from jax.experimental import pallas as pl
from jax.experimental.pallas import tpu as pltpu
from jax.experimental.pallas import tpu_sc as plsc

N = 1048576
NBLK = N // 128
LANES = 16
B = 256
BBLK = B // 128
NG = B // LANES
IDX_CHUNK = 128

PLANE_ROWS = 2048 * 2048 // 8

_T1 = (9, 128, 8)
_T2 = (15, 64, 4)

_COLMAP = []
for _c in range(3):
    _COLMAP.append((0, _c, _c * 9 + 0))
    for _k in range(3):
        _COLMAP.append((1, 3 * _c + _k, _c * 9 + 1 + _k))
    for _k in range(5):
        _COLMAP.append((2, 5 * _c + _k, _c * 9 + 4 + _k))


def _scratch_set():
    return [
        pltpu.VMEM((BBLK, 2, 128), jnp.float32),
        pltpu.VMEM((3 * B,), jnp.int32),
        pltpu.VMEM((B,), jnp.int32),
        pltpu.VMEM((B,), jnp.int32),
        pltpu.VMEM((B,), jnp.int32),
        pltpu.VMEM((3 * B, 8), jnp.float32),
        pltpu.VMEM((B, 16), jnp.float32),
        pltpu.VMEM((B, 16), jnp.float32),
        pltpu.VMEM((9, BBLK, 4, 128), jnp.float32),
        pltpu.SemaphoreType.DMA,
        pltpu.SemaphoreType.DMA,
    ]


def kernel(uv_coords, tex0, tex1, tex2):
    t0_rows = (tex0.transpose(2, 0, 1)
               .reshape(3, 256, 8, 16, 128)
               .transpose(0, 1, 3, 2, 4)
               .reshape(3 * PLANE_ROWS, 8))
    t1_planes = (tex1.transpose(2, 0, 1)
                 .reshape(9, 128, 8, 8, 128)
                 .transpose(0, 1, 3, 2, 4)
                 .reshape(9, 128, 8, 1024))
    t2_planes = (tex2.transpose(2, 0, 1)
                 .reshape(15, 64, 8, 4, 128)
                 .transpose(0, 1, 3, 2, 4)
                 .reshape(15, 64, 4, 1024))
    uv_blk = uv_coords.reshape(NBLK, 128, 2).transpose(0, 2, 1)

    info = plsc.get_sparse_core_info()
    nc, ns = info.num_cores, info.num_subcores
    nw = nc * ns
    pts_per_tile = N // nw
    n_chunks = pts_per_tile // B
    n_pairs = n_chunks // 2

    @functools.partial(
        pl.kernel,
        out_type=[
            jax.ShapeDtypeStruct((9, NBLK, 4, 128), jnp.float32),
            jax.ShapeDtypeStruct((1024 * 1024, 16), jnp.float32),
            jax.ShapeDtypeStruct((512 * 512, 16), jnp.float32),
        ],
        mesh=plsc.VectorSubcoreMesh(core_axis_name="c", subcore_axis_name="s"),
        compiler_params=pltpu.CompilerParams(
            needs_layout_passes=False, use_tc_tiling_on_sc=False),
        scratch_types=_scratch_set() + _scratch_set() + [
            pltpu.VMEM((15, 1024), jnp.float32),
            pltpu.VMEM((15, 1024), jnp.float32),
            pltpu.VMEM((1024, 16), jnp.float32),
            pltpu.VMEM((1024, 16), jnp.float32),
            pltpu.SemaphoreType.DMA,
            pltpu.SemaphoreType.DMA,
            pltpu.SemaphoreType.DMA,
            pltpu.SemaphoreType.DMA,
        ],
    )
    def sc_kernel(uv_hbm, t0_hbm, t1pl_hbm, t2pl_hbm,
                  out_hbm, t1p_hbm, t2p_hbm, *scr):
        bufA, bufB = scr[:11], scr[11:22]
        rinA, rinB, routA, routB, rsemA, rsemB, rosemA, rosemB = scr[22:]
        wid = lax.axis_index("s") * nc + lax.axis_index("c")
        sid = lax.axis_index("s")
        iota = lax.iota(jnp.int32, LANES)
        zeros = jnp.zeros((LANES,), jnp.int32)

        def repack(planes_hbm, packed_hbm, nchan, nty, ntx):
            nblocks = nty * ntx
            per_tile = nblocks // ns
            texw = ntx * 128
            lo = sid * per_tile

            def fire_in(bi, rin, rsem):
                ty = bi // ntx
                tx = bi - ty * ntx
                pltpu.async_copy(planes_hbm.at[:, ty, tx],
                                 rin.at[pl.ds(0, nchan)], rsem)

            def wait_in(bi, rin, rsem):
                ty = bi // ntx
                tx = bi - ty * ntx
                pltpu.make_async_copy(planes_hbm.at[:, ty, tx],
                                      rin.at[pl.ds(0, nchan)], rsem).wait()

            def drain_outs(rout, rosem):
                for sy in range(8):
                    pltpu.make_async_copy(
                        rout.at[pl.ds(sy * 128, 128)],
                        packed_hbm.at[pl.ds(sy * 128, 128)], rosem).wait()

            def process(bi, rin, rout, rosem):
                ty = bi // ntx
                tx = bi - ty * ntx

                @plsc.parallel_loop(0, 64, 1, unroll=2)
                def fill(g):
                    vrow = iota + g * LANES
                    for c in range(nchan):
                        val = rin[c, pl.ds(g * LANES, LANES)]
                        plsc.store_scatter(rout, [vrow, zeros + c], val)

                row0 = (ty * 8) * texw + tx * 128
                for sy in range(8):
                    pltpu.async_copy(
                        rout.at[pl.ds(sy * 128, 128)],
                        packed_hbm.at[pl.ds(row0 + sy * texw, 128)], rosem)

            fire_in(lo, rinA, rsemA)

            def rpair(cp, _):
                ba = lo + cp * 2
                bb = ba + 1
                fire_in(bb, rinB, rsemB)
                wait_in(ba, rinA, rsemA)

                @pl.when(cp > 0)
                def _da():
                    drain_outs(routA, rosemA)

                process(ba, rinA, routA, rosemA)

                @pl.when(cp < per_tile // 2 - 1)
                def _next():
                    fire_in(ba + 2, rinA, rsemA)

                wait_in(bb, rinB, rsemB)

                @pl.when(cp > 0)
                def _db():
                    drain_outs(routB, rosemB)

                process(bb, rinB, routB, rosemB)
                return _

            lax.fori_loop(0, per_tile // 2, rpair, None)
            drain_outs(routA, rosemA)
            drain_outs(routB, rosemB)

        repack(t1pl_hbm, t1p_hbm, *_T1)
        repack(t2pl_hbm, t2p_hbm, *_T2)
        plsc.subcore_barrier()

        def chunk_base(ci):
            return wid * pts_per_tile + ci * B

        def fire(ci, buf):
            (uv_v, i0_v, i1_v, i2_v, bo0_v,
             g0_v, g1_v, g2_v, out_v, sem_g, sem_o) = buf
            blk0 = chunk_base(ci) // 128
            pltpu.sync_copy(uv_hbm.at[pl.ds(blk0, BBLK)], uv_v)

            @plsc.parallel_loop(0, NG, 1, unroll=2)
            def gen_body(g):
                q = iota + g * LANES
                qb = q >> 7
                ql = q & 127
                q2 = q << 1
                q3 = q2 + q
                u = plsc.load_gather(uv_v, [qb, zeros, ql])
                v = plsc.load_gather(uv_v, [qb, zeros + 1, ql])
                ix = jnp.clip((u * 2048.0).astype(jnp.int32), 0, 2047)
                iy = jnp.clip((v * 2048.0).astype(jnp.int32), 0, 2047)
                w = ((((iy >> 3) << 4) + (ix >> 7)) << 10) \
                    + ((iy & 7) << 7) + (ix & 127)
                r0 = w >> 3
                plsc.store_scatter(i0_v, [q3], r0)
                plsc.store_scatter(i0_v, [q3 + 1], r0 + PLANE_ROWS)
                plsc.store_scatter(i0_v, [q3 + 2], r0 + 2 * PLANE_ROWS)
                plsc.store_scatter(bo0_v, [q], ix & 7)
                plsc.store_scatter(i1_v, [q],
                                   ((iy >> 1) << 10) + (ix >> 1))
                plsc.store_scatter(i2_v, [q],
                                   ((iy >> 2) << 9) + (ix >> 2))

            for i in range(3 * B // IDX_CHUNK):
                sl = pl.ds(i * IDX_CHUNK, IDX_CHUNK)
                pltpu.async_copy(t0_hbm.at[i0_v.at[sl]], g0_v.at[sl], sem_g)
            for i in range(B // IDX_CHUNK):
                sl = pl.ds(i * IDX_CHUNK, IDX_CHUNK)
                pltpu.async_copy(t1p_hbm.at[i1_v.at[sl]], g1_v.at[sl], sem_g)
                pltpu.async_copy(t2p_hbm.at[i2_v.at[sl]], g2_v.at[sl], sem_g)

        def drain_gathers(buf):
            (uv_v, i0_v, i1_v, i2_v, bo0_v,
             g0_v, g1_v, g2_v, out_v, sem_g, sem_o) = buf
            for i in range(3 * B // IDX_CHUNK):
                sl = pl.ds(i * IDX_CHUNK, IDX_CHUNK)
                pltpu.make_async_copy(
                    t0_hbm.at[i0_v.at[sl]], g0_v.at[sl], sem_g).wait()
            for i in range(B // IDX_CHUNK):
                sl = pl.ds(i * IDX_CHUNK, IDX_CHUNK)
                pltpu.make_async_copy(
                    t1p_hbm.at[i1_v.at[sl]], g1_v.at[sl], sem_g).wait()
                pltpu.make_async_copy(
                    t2p_hbm.at[i2_v.at[sl]], g2_v.at[sl], sem_g).wait()

        def drain_out(ci, buf):
            out_v, sem_o = buf[8], buf[10]
            blk0 = chunk_base(ci) // 128
            pltpu.make_async_copy(
                out_v, out_hbm.at[:, pl.ds(blk0, BBLK)], sem_o).wait()

        def shuffle_and_out(ci, buf):
            (uv_v, i0_v, i1_v, i2_v, bo0_v,
             g0_v, g1_v, g2_v, out_v, sem_g, sem_o) = buf
            blk0 = chunk_base(ci) // 128

            @plsc.parallel_loop(0, NG, 1, unroll=2)
            def shuf_body(g):
                q = iota + g * LANES
                qb = q >> 7
                ql = q & 127
                q3 = (q << 1) + q
                x7 = plsc.load_gather(bo0_v, [q])
                for s, col, j in _COLMAP:
                    if s == 0:
                        val = plsc.load_gather(g0_v, [q3 + col, x7])
                    else:
                        val = plsc.load_gather(
                            (g1_v, g2_v)[s - 1], [q, zeros + col])
                    plsc.store_scatter(
                        out_v,
                        [zeros + (j % 9), qb, zeros + (j // 9), ql], val)

            pltpu.async_copy(out_v, out_hbm.at[:, pl.ds(blk0, BBLK)], sem_o)

        fire(0, bufA)

        def pair_body(cp, _):
            ca = cp * 2
            cb = ca + 1
            fire(cb, bufB)
            drain_gathers(bufA)

            @pl.when(cp > 0)
            def _older_a():
                drain_out(ca, bufA)

            shuffle_and_out(ca, bufA)

            @pl.when(cp < n_pairs - 1)
            def _next_a():
                fire(ca + 2, bufA)

            drain_gathers(bufB)

            @pl.when(cp > 0)
            def _older_b():
                drain_out(cb, bufB)

            shuffle_and_out(cb, bufB)
            return _

        lax.fori_loop(0, n_pairs, pair_body, None)
        drain_out(0, bufA)
        drain_out(0, bufB)

    out, _, _ = sc_kernel(uv_blk, t0_rows, t1_planes, t2_planes)
    return (out.transpose(1, 3, 2, 0)
            .reshape(N, 4, 9)[:, :3, :])


# speedup vs baseline: 2.4080x; 1.0919x over previous
import functools

import jax
import jax.numpy as jnp
from jax import lax
from jax.experimental import pallas as pl
from jax.experimental.pallas import tpu as pltpu
from jax.experimental.pallas import tpu_sc as plsc

N = 1048576
NBLK = N // 128
LANES = 16
B = 256
BBLK = B // 128
NG = B // LANES
IDX_CHUNK = 128

PLANE_ROWS = 2048 * 2048 // 8

_T1 = (9, 128, 8)
_T2 = (15, 64, 4)

_COLMAP = []
for _c in range(3):
    _COLMAP.append((0, _c, _c * 9 + 0))
    for _k in range(3):
        _COLMAP.append((1, 3 * _c + _k, _c * 9 + 1 + _k))
    for _k in range(5):
        _COLMAP.append((2, 5 * _c + _k, _c * 9 + 4 + _k))


def _scratch_set():
    return [
        pltpu.VMEM((BBLK, 2, 128), jnp.float32),
        pltpu.VMEM((3 * B,), jnp.int32),
        pltpu.VMEM((B,), jnp.int32),
        pltpu.VMEM((B,), jnp.int32),
        pltpu.VMEM((B,), jnp.int32),
        pltpu.VMEM((3 * B, 8), jnp.float32),
        pltpu.VMEM((B, 16), jnp.float32),
        pltpu.VMEM((B, 16), jnp.float32),
        pltpu.VMEM((9, BBLK, 3, 128), jnp.float32),
        pltpu.SemaphoreType.DMA,
        pltpu.SemaphoreType.DMA,
        pltpu.SemaphoreType.DMA,
    ]


def kernel(uv_coords, tex0, tex1, tex2):
    t0_rows = (tex0.transpose(2, 0, 1)
               .reshape(3, 256, 8, 16, 128)
               .transpose(0, 1, 3, 2, 4)
               .reshape(3 * PLANE_ROWS, 8))
    t1_planes = (tex1.transpose(2, 0, 1)
                 .reshape(9, 128, 8, 8, 128)
                 .transpose(0, 1, 3, 2, 4)
                 .reshape(9, 128, 8, 1024))
    t2_planes = (tex2.transpose(2, 0, 1)
                 .reshape(15, 64, 8, 4, 128)
                 .transpose(0, 1, 3, 2, 4)
                 .reshape(15, 64, 4, 1024))
    uv_blk = uv_coords.reshape(NBLK, 128, 2).transpose(0, 2, 1)

    info = plsc.get_sparse_core_info()
    nc, ns = info.num_cores, info.num_subcores
    nw = nc * ns
    pts_per_tile = N // nw
    n_chunks = pts_per_tile // B
    n_pairs = n_chunks // 2

    @functools.partial(
        pl.kernel,
        out_type=[
            jax.ShapeDtypeStruct((9, NBLK, 4, 128), jnp.float32),
            jax.ShapeDtypeStruct((1024 * 1024, 16), jnp.float32),
            jax.ShapeDtypeStruct((512 * 512, 16), jnp.float32),
        ],
        mesh=plsc.VectorSubcoreMesh(core_axis_name="c", subcore_axis_name="s"),
        compiler_params=pltpu.CompilerParams(
            needs_layout_passes=False, use_tc_tiling_on_sc=False),
        scratch_types=_scratch_set() + _scratch_set() + [
            pltpu.VMEM((15, 1024), jnp.float32),
            pltpu.VMEM((15, 1024), jnp.float32),
            pltpu.VMEM((1024, 16), jnp.float32),
            pltpu.VMEM((1024, 16), jnp.float32),
            pltpu.SemaphoreType.DMA,
            pltpu.SemaphoreType.DMA,
            pltpu.SemaphoreType.DMA,
            pltpu.SemaphoreType.DMA,
        ],
    )
    def sc_kernel(uv_hbm, t0_hbm, t1pl_hbm, t2pl_hbm,
                  out_hbm, t1p_hbm, t2p_hbm, *scr):
        bufA, bufB = scr[:12], scr[12:24]
        rinA, rinB, routA, routB, rsemA, rsemB, rosemA, rosemB = scr[24:]
        wid = lax.axis_index("s") * nc + lax.axis_index("c")
        sid = lax.axis_index("s")
        iota = lax.iota(jnp.int32, LANES)
        zeros = jnp.zeros((LANES,), jnp.int32)

        def repack(planes_hbm, packed_hbm, nchan, nty, ntx):
            nblocks = nty * ntx
            per_tile = nblocks // ns
            texw = ntx * 128
            lo = sid * per_tile

            def fire_in(bi, rin, rsem):
                ty = bi // ntx
                tx = bi - ty * ntx
                pltpu.async_copy(planes_hbm.at[:, ty, tx],
                                 rin.at[pl.ds(0, nchan)], rsem)

            def wait_in(bi, rin, rsem):
                ty = bi // ntx
                tx = bi - ty * ntx
                pltpu.make_async_copy(planes_hbm.at[:, ty, tx],
                                      rin.at[pl.ds(0, nchan)], rsem).wait()

            def drain_outs(rout, rosem):
                for sy in range(8):
                    pltpu.make_async_copy(
                        rout.at[pl.ds(sy * 128, 128)],
                        packed_hbm.at[pl.ds(sy * 128, 128)], rosem).wait()

            def process(bi, rin, rout, rosem):
                ty = bi // ntx
                tx = bi - ty * ntx

                @plsc.parallel_loop(0, 64, 1, unroll=2)
                def fill(g):
                    vrow = iota + g * LANES
                    for c in range(nchan):
                        val = rin[c, pl.ds(g * LANES, LANES)]
                        plsc.store_scatter(rout, [vrow, zeros + c], val)

                row0 = (ty * 8) * texw + tx * 128
                for sy in range(8):
                    pltpu.async_copy(
                        rout.at[pl.ds(sy * 128, 128)],
                        packed_hbm.at[pl.ds(row0 + sy * texw, 128)], rosem)

            fire_in(lo, rinA, rsemA)

            def rpair(cp, _):
                ba = lo + cp * 2
                bb = ba + 1
                fire_in(bb, rinB, rsemB)
                wait_in(ba, rinA, rsemA)

                @pl.when(cp > 0)
                def _da():
                    drain_outs(routA, rosemA)

                process(ba, rinA, routA, rosemA)

                @pl.when(cp < per_tile // 2 - 1)
                def _next():
                    fire_in(ba + 2, rinA, rsemA)

                wait_in(bb, rinB, rsemB)

                @pl.when(cp > 0)
                def _db():
                    drain_outs(routB, rosemB)

                process(bb, rinB, routB, rosemB)
                return _

            lax.fori_loop(0, per_tile // 2, rpair, None)
            drain_outs(routA, rosemA)
            drain_outs(routB, rosemB)

        repack(t1pl_hbm, t1p_hbm, *_T1)
        repack(t2pl_hbm, t2p_hbm, *_T2)
        plsc.subcore_barrier()

        def chunk_base(ci):
            return wid * pts_per_tile + ci * B

        def fire_uv(ci, buf):
            uv_v, sem_u = buf[0], buf[11]
            blk0 = chunk_base(ci) // 128
            pltpu.async_copy(uv_hbm.at[pl.ds(blk0, BBLK)], uv_v, sem_u)

        def fire(ci, buf):
            (uv_v, i0_v, i1_v, i2_v, bo0_v,
             g0_v, g1_v, g2_v, out_v, sem_g, sem_o, sem_u) = buf
            blk0 = chunk_base(ci) // 128
            pltpu.make_async_copy(
                uv_hbm.at[pl.ds(blk0, BBLK)], uv_v, sem_u).wait()

            @plsc.parallel_loop(0, NG, 1, unroll=2)
            def gen_body(g):
                q = iota + g * LANES
                qb = q >> 7
                ql = q & 127
                q2 = q << 1
                q3 = q2 + q
                u = plsc.load_gather(uv_v, [qb, zeros, ql])
                v = plsc.load_gather(uv_v, [qb, zeros + 1, ql])
                ix = jnp.clip((u * 2048.0).astype(jnp.int32), 0, 2047)
                iy = jnp.clip((v * 2048.0).astype(jnp.int32), 0, 2047)
                w = ((((iy >> 3) << 4) + (ix >> 7)) << 10) \
                    + ((iy & 7) << 7) + (ix & 127)
                r0 = w >> 3
                plsc.store_scatter(i0_v, [q3], r0)
                plsc.store_scatter(i0_v, [q3 + 1], r0 + PLANE_ROWS)
                plsc.store_scatter(i0_v, [q3 + 2], r0 + 2 * PLANE_ROWS)
                plsc.store_scatter(bo0_v, [q], ix & 7)
                plsc.store_scatter(i1_v, [q],
                                   ((iy >> 1) << 10) + (ix >> 1))
                plsc.store_scatter(i2_v, [q],
                                   ((iy >> 2) << 9) + (ix >> 2))

            for i in range(3 * B // IDX_CHUNK):
                sl = pl.ds(i * IDX_CHUNK, IDX_CHUNK)
                pltpu.async_copy(t0_hbm.at[i0_v.at[sl]], g0_v.at[sl], sem_g)
            for i in range(B // IDX_CHUNK):
                sl = pl.ds(i * IDX_CHUNK, IDX_CHUNK)
                pltpu.async_copy(t1p_hbm.at[i1_v.at[sl]], g1_v.at[sl], sem_g)
                pltpu.async_copy(t2p_hbm.at[i2_v.at[sl]], g2_v.at[sl], sem_g)

            @pl.when(ci + 2 < n_chunks)
            def _uv_next():
                fire_uv(ci + 2, buf)

        def drain_gathers(buf):
            (uv_v, i0_v, i1_v, i2_v, bo0_v,
             g0_v, g1_v, g2_v, out_v, sem_g, sem_o, sem_u) = buf
            for i in range(3 * B // IDX_CHUNK):
                sl = pl.ds(i * IDX_CHUNK, IDX_CHUNK)
                pltpu.make_async_copy(
                    t0_hbm.at[i0_v.at[sl]], g0_v.at[sl], sem_g).wait()
            for i in range(B // IDX_CHUNK):
                sl = pl.ds(i * IDX_CHUNK, IDX_CHUNK)
                pltpu.make_async_copy(
                    t1p_hbm.at[i1_v.at[sl]], g1_v.at[sl], sem_g).wait()
                pltpu.make_async_copy(
                    t2p_hbm.at[i2_v.at[sl]], g2_v.at[sl], sem_g).wait()

        def drain_out(ci, buf):
            out_v, sem_o = buf[8], buf[10]
            blk0 = chunk_base(ci) // 128
            pltpu.make_async_copy(
                out_v, out_hbm.at[:, pl.ds(blk0, BBLK), pl.ds(0, 3)],
                sem_o).wait()

        def shuffle_and_out(ci, buf):
            (uv_v, i0_v, i1_v, i2_v, bo0_v,
             g0_v, g1_v, g2_v, out_v, sem_g, sem_o, sem_u) = buf
            blk0 = chunk_base(ci) // 128

            @plsc.parallel_loop(0, NG, 1, unroll=2)
            def shuf_body(g):
                q = iota + g * LANES
                qb = q >> 7
                ql = q & 127
                q3 = (q << 1) + q
                x7 = plsc.load_gather(bo0_v, [q])
                for s, col, j in _COLMAP:
                    if s == 0:
                        val = plsc.load_gather(g0_v, [q3 + col, x7])
                    else:
                        val = plsc.load_gather(
                            (g1_v, g2_v)[s - 1], [q, zeros + col])
                    plsc.store_scatter(
                        out_v,
                        [zeros + (j % 9), qb, zeros + (j // 9), ql], val)

            pltpu.async_copy(
                out_v, out_hbm.at[:, pl.ds(blk0, BBLK), pl.ds(0, 3)], sem_o)

        fire_uv(0, bufA)
        fire_uv(1, bufB)
        fire(0, bufA)

        def pair_body(cp, _):
            ca = cp * 2
            cb = ca + 1
            fire(cb, bufB)
            drain_gathers(bufA)

            @pl.when(cp > 0)
            def _older_a():
                drain_out(ca, bufA)

            shuffle_and_out(ca, bufA)

            @pl.when(cp < n_pairs - 1)
            def _next_a():
                fire(ca + 2, bufA)

            drain_gathers(bufB)

            @pl.when(cp > 0)
            def _older_b():
                drain_out(cb, bufB)

            shuffle_and_out(cb, bufB)
            return _

        lax.fori_loop(0, n_pairs, pair_body, None)
        drain_out(0, bufA)
        drain_out(0, bufB)

    out, _, _ = sc_kernel(uv_blk, t0_rows, t1_planes, t2_planes)
    return (out.transpose(1, 3, 2, 0)
            .reshape(N, 4, 9)[:, :3, :])
